# Initial kernel scaffold; baseline (speedup 1.0000x reference)
import jax
import jax.numpy as jnp
from jax.experimental import pallas as pl


def kernel(x, num0, num1, num2, W0, b0, W1, g1, e1, W2, g2, e2, W3, g3, e3, W4, g4, e4, W5, g5, e5, W6, g6, e6, W7, g7, e7):
    raise NotImplementedError("write your pallas kernel here")


# speedup vs baseline: 1.0018x; 1.0018x over previous
import functools

import jax
import jax.numpy as jnp
from jax.experimental import pallas as pl
from jax.experimental.pallas import tpu as pltpu


def _lrelu(x):
    return jnp.where(x > 0, x, 0.2 * x)


def _square_distance(src, dst):
    d = -2.0 * jnp.matmul(src, jnp.transpose(dst, (0, 2, 1)))
    d = d + jnp.sum(src ** 2, -1)[:, :, None]
    d = d + jnp.sum(dst ** 2, -1)[:, None, :]
    return d


def _knn_idx(k, coor_k, coor_q):
    d = _square_distance(coor_q, coor_k)
    _, idx = jax.lax.top_k(-d, k)
    return idx


def _fps(xyz, npoint):
    B, N, _ = xyz.shape
    def body(i, state):
        idxs, dists, far = state
        idxs = idxs.at[:, i].set(far)
        cent = jnp.take_along_axis(xyz, far[:, None, None], axis=1)
        d = jnp.sum((xyz - cent) ** 2, -1)
        dists = jnp.minimum(dists, d)
        far = jnp.argmax(dists, -1).astype(jnp.int32)
        return (idxs, dists, far)
    state = (jnp.zeros((B, npoint), jnp.int32),
             jnp.full((B, N), 1e10, jnp.float32),
             jnp.zeros((B,), jnp.int32))
    idxs, _, _ = jax.lax.fori_loop(0, npoint, body, state)
    return idxs


def _graph_feature(x_q, x_k, idx):
    B = x_q.shape[0]
    xk_t = jnp.transpose(x_k, (0, 2, 1))
    g = xk_t[jnp.arange(B)[:, None, None], idx]
    feat = jnp.transpose(g, (0, 3, 1, 2))
    xq = x_q[:, :, :, None]
    return jnp.concatenate([feat - xq, jnp.broadcast_to(xq, feat.shape)], axis=1)


def _conv_gn_lrelu(x, W, gamma, beta, groups=4, eps=1e-5):
    y = jnp.einsum('oi,bink->bonk', W, x)
    B, C, N, K = y.shape
    yg = y.reshape(B, groups, C // groups, N, K)
    m = jnp.mean(yg, axis=(2, 3, 4), keepdims=True)
    v = jnp.var(yg, axis=(2, 3, 4), keepdims=True)
    yg = (yg - m) / jnp.sqrt(v + eps)
    y = yg.reshape(B, C, N, K) * gamma[None, :, None, None] + beta[None, :, None, None]
    return _lrelu(y)


def _stem_kernel(x_ref, w_ref, b_ref, o_ref):
    x = x_ref[0]
    w = w_ref[...]
    b = b_ref[...]
    o_ref[0] = jax.lax.dot_general(
        w, x, (((1,), (0,)), ((), ())),
        preferred_element_type=jnp.float32) + b


def _stem(xyzT, W0, b0):
    B, _, N = xyzT.shape
    return pl.pallas_call(
        _stem_kernel,
        grid=(B,),
        in_specs=[
            pl.BlockSpec((1, 3, N), lambda b: (b, 0, 0)),
            pl.BlockSpec((8, 3), lambda b: (0, 0)),
            pl.BlockSpec((8, 1), lambda b: (0, 0)),
        ],
        out_specs=pl.BlockSpec((1, 8, N), lambda b: (b, 0, 0)),
        out_shape=jax.ShapeDtypeStruct((B, 8, N), jnp.float32),
    )(xyzT, W0, b0[:, None])


def kernel(x, num0, num1, num2, W0, b0, W1, g1, e1, W2, g2, e2, W3, g3, e3,
           W4, g4, e4, W5, g5, e5, W6, g6, e6, W7, g7, e7):
    k = 16
    coor = x
    i0 = _knn_idx(k, coor, coor)
    fi1 = _fps(coor, 512)
    pts1 = jnp.take_along_axis(coor, fi1[:, :, None], axis=1)
    i1 = _knn_idx(k, coor, pts1)
    i2 = _knn_idx(k, pts1, pts1)
    fi2 = _fps(pts1, 256)
    pts2 = jnp.take_along_axis(pts1, fi2[:, :, None], axis=1)
    i3 = _knn_idx(k, pts1, pts2)
    i4 = _knn_idx(k, pts2, pts2)
    fi3 = _fps(pts2, 128)
    pts3 = jnp.take_along_axis(pts2, fi3[:, :, None], axis=1)
    i5 = _knn_idx(k, pts2, pts3)
    i6 = _knn_idx(k, pts3, pts3)

    xyzT = jnp.transpose(x, (0, 2, 1))
    f = _stem(xyzT, W0, b0)
    f = _conv_gn_lrelu(_graph_feature(f, f, i0), W1, g1, e1).max(axis=-1)
    f_q = jnp.take_along_axis(f, fi1[:, None, :], axis=2)
    f = _conv_gn_lrelu(_graph_feature(f_q, f, i1), W2, g2, e2).max(axis=-1)
    s1 = _conv_gn_lrelu(_graph_feature(f, f, i2), W3, g3, e3).max(axis=-1)
    f_q = jnp.take_along_axis(s1, fi2[:, None, :], axis=2)
    f = _conv_gn_lrelu(_graph_feature(f_q, s1, i3), W4, g4, e4).max(axis=-1)
    s2 = _conv_gn_lrelu(_graph_feature(f, f, i4), W5, g5, e5).max(axis=-1)
    f_q = jnp.take_along_axis(s2, fi3[:, None, :], axis=2)
    f = _conv_gn_lrelu(_graph_feature(f_q, s2, i5), W6, g6, e6).max(axis=-1)
    s3 = _conv_gn_lrelu(_graph_feature(f, f, i6), W7, g7, e7).max(axis=-1)

    encoded = jnp.transpose(s3, (0, 2, 1))
    return (s1, s2, s3, pts1, pts2, pts3, encoded)


# speedup vs baseline: 1.3249x; 1.3224x over previous
import functools

import jax
import jax.numpy as jnp
from jax.experimental import pallas as pl
from jax.experimental.pallas import tpu as pltpu


def _lrelu(x):
    return jnp.where(x > 0, x, 0.2 * x)


def _square_distance(src, dst):
    d = -2.0 * jnp.matmul(src, jnp.transpose(dst, (0, 2, 1)))
    d = d + jnp.sum(src ** 2, -1)[:, :, None]
    d = d + jnp.sum(dst ** 2, -1)[:, None, :]
    return d


def _knn_idx(k, coor_k, coor_q):
    d = _square_distance(coor_q, coor_k)
    _, idx = jax.lax.top_k(-d, k)
    return idx


def _fps_stage(X, Y, Z, npoint, fi_ref, px_ref, py_ref, pz_ref,
               dists_ref, far_ref):
    B, N = X.shape
    iota = jax.lax.broadcasted_iota(jnp.int32, (B, N), 1)
    oiota = jax.lax.broadcasted_iota(jnp.int32, (B, npoint), 1)

    dists_ref[:, :N] = jnp.full((B, N), 1e10, jnp.float32)
    far_ref[...] = jnp.zeros((B, 128), jnp.int32)

    def body(i, _):
        far = far_ref[:, 0:1]
        slot = oiota == i
        fi_ref[...] = jnp.where(slot, far + jnp.zeros_like(oiota), fi_ref[...])
        sel = iota == far
        cx = jnp.sum(jnp.where(sel, X, 0.0), axis=1, keepdims=True)
        cy = jnp.sum(jnp.where(sel, Y, 0.0), axis=1, keepdims=True)
        cz = jnp.sum(jnp.where(sel, Z, 0.0), axis=1, keepdims=True)
        zf = jnp.zeros((B, npoint), jnp.float32)
        px_ref[...] = jnp.where(slot, cx + zf, px_ref[...])
        py_ref[...] = jnp.where(slot, cy + zf, py_ref[...])
        pz_ref[...] = jnp.where(slot, cz + zf, pz_ref[...])
        dx = X - cx
        dy = Y - cy
        dz = Z - cz
        d = (dx * dx + dz * dz) + dy * dy
        dists = jnp.minimum(dists_ref[:, :N], d)
        dists_ref[:, :N] = dists
        m = jnp.max(dists, axis=1, keepdims=True)
        far_ref[:, 0:1] = jnp.min(jnp.where(dists == m, iota, N),
                                  axis=1, keepdims=True)
        return 0

    jax.lax.fori_loop(0, npoint, body, 0)


def _fps_kernel(x_ref, y_ref, z_ref,
                fi1_ref, p1x_ref, p1y_ref, p1z_ref,
                fi2_ref, p2x_ref, p2y_ref, p2z_ref,
                fi3_ref, p3x_ref, p3y_ref, p3z_ref,
                dists_ref, far_ref):
    _fps_stage(x_ref[...], y_ref[...], z_ref[...], 512,
               fi1_ref, p1x_ref, p1y_ref, p1z_ref, dists_ref, far_ref)
    _fps_stage(p1x_ref[...], p1y_ref[...], p1z_ref[...], 256,
               fi2_ref, p2x_ref, p2y_ref, p2z_ref, dists_ref, far_ref)
    _fps_stage(p2x_ref[...], p2y_ref[...], p2z_ref[...], 128,
               fi3_ref, p3x_ref, p3y_ref, p3z_ref, dists_ref, far_ref)


def _fps_all(x):
    B, N, _ = x.shape
    xT = jnp.transpose(x, (0, 2, 1))
    X, Y, Z = xT[:, 0], xT[:, 1], xT[:, 2]
    outs = pl.pallas_call(
        _fps_kernel,
        out_shape=[
            jax.ShapeDtypeStruct((B, 512), jnp.int32),
            jax.ShapeDtypeStruct((B, 512), jnp.float32),
            jax.ShapeDtypeStruct((B, 512), jnp.float32),
            jax.ShapeDtypeStruct((B, 512), jnp.float32),
            jax.ShapeDtypeStruct((B, 256), jnp.int32),
            jax.ShapeDtypeStruct((B, 256), jnp.float32),
            jax.ShapeDtypeStruct((B, 256), jnp.float32),
            jax.ShapeDtypeStruct((B, 256), jnp.float32),
            jax.ShapeDtypeStruct((B, 128), jnp.int32),
            jax.ShapeDtypeStruct((B, 128), jnp.float32),
            jax.ShapeDtypeStruct((B, 128), jnp.float32),
            jax.ShapeDtypeStruct((B, 128), jnp.float32),
        ],
        scratch_shapes=[
            pltpu.VMEM((B, N), jnp.float32),
            pltpu.VMEM((B, 128), jnp.int32),
        ],
    )(X, Y, Z)
    fi1, p1x, p1y, p1z, fi2, p2x, p2y, p2z, fi3, p3x, p3y, p3z = outs
    pts1 = jnp.stack([p1x, p1y, p1z], axis=-1)
    pts2 = jnp.stack([p2x, p2y, p2z], axis=-1)
    pts3 = jnp.stack([p3x, p3y, p3z], axis=-1)
    return fi1, fi2, fi3, pts1, pts2, pts3


def _graph_feature(x_q, x_k, idx):
    B = x_q.shape[0]
    xk_t = jnp.transpose(x_k, (0, 2, 1))
    g = xk_t[jnp.arange(B)[:, None, None], idx]
    feat = jnp.transpose(g, (0, 3, 1, 2))
    xq = x_q[:, :, :, None]
    return jnp.concatenate([feat - xq, jnp.broadcast_to(xq, feat.shape)], axis=1)


def _conv_gn_lrelu(x, W, gamma, beta, groups=4, eps=1e-5):
    y = jnp.einsum('oi,bink->bonk', W, x)
    B, C, N, K = y.shape
    yg = y.reshape(B, groups, C // groups, N, K)
    m = jnp.mean(yg, axis=(2, 3, 4), keepdims=True)
    v = jnp.var(yg, axis=(2, 3, 4), keepdims=True)
    yg = (yg - m) / jnp.sqrt(v + eps)
    y = yg.reshape(B, C, N, K) * gamma[None, :, None, None] + beta[None, :, None, None]
    return _lrelu(y)


def _stem_kernel(x_ref, w_ref, b_ref, o_ref):
    x = x_ref[0]
    w = w_ref[...]
    b = b_ref[...]
    o_ref[0] = jax.lax.dot_general(
        w, x, (((1,), (0,)), ((), ())),
        preferred_element_type=jnp.float32) + b


def _stem(xyzT, W0, b0):
    B, _, N = xyzT.shape
    return pl.pallas_call(
        _stem_kernel,
        grid=(B,),
        in_specs=[
            pl.BlockSpec((1, 3, N), lambda b: (b, 0, 0)),
            pl.BlockSpec((8, 3), lambda b: (0, 0)),
            pl.BlockSpec((8, 1), lambda b: (0, 0)),
        ],
        out_specs=pl.BlockSpec((1, 8, N), lambda b: (b, 0, 0)),
        out_shape=jax.ShapeDtypeStruct((B, 8, N), jnp.float32),
    )(xyzT, W0, b0[:, None])


def kernel(x, num0, num1, num2, W0, b0, W1, g1, e1, W2, g2, e2, W3, g3, e3,
           W4, g4, e4, W5, g5, e5, W6, g6, e6, W7, g7, e7):
    k = 16
    coor = x
    fi1, fi2, fi3, pts1, pts2, pts3 = _fps_all(coor)
    i0 = _knn_idx(k, coor, coor)
    i1 = _knn_idx(k, coor, pts1)
    i2 = _knn_idx(k, pts1, pts1)
    i3 = _knn_idx(k, pts1, pts2)
    i4 = _knn_idx(k, pts2, pts2)
    i5 = _knn_idx(k, pts2, pts3)
    i6 = _knn_idx(k, pts3, pts3)

    xyzT = jnp.transpose(x, (0, 2, 1))
    f = _stem(xyzT, W0, b0)
    f = _conv_gn_lrelu(_graph_feature(f, f, i0), W1, g1, e1).max(axis=-1)
    f_q = jnp.take_along_axis(f, fi1[:, None, :], axis=2)
    f = _conv_gn_lrelu(_graph_feature(f_q, f, i1), W2, g2, e2).max(axis=-1)
    s1 = _conv_gn_lrelu(_graph_feature(f, f, i2), W3, g3, e3).max(axis=-1)
    f_q = jnp.take_along_axis(s1, fi2[:, None, :], axis=2)
    f = _conv_gn_lrelu(_graph_feature(f_q, s1, i3), W4, g4, e4).max(axis=-1)
    s2 = _conv_gn_lrelu(_graph_feature(f, f, i4), W5, g5, e5).max(axis=-1)
    f_q = jnp.take_along_axis(s2, fi3[:, None, :], axis=2)
    f = _conv_gn_lrelu(_graph_feature(f_q, s2, i5), W6, g6, e6).max(axis=-1)
    s3 = _conv_gn_lrelu(_graph_feature(f, f, i6), W7, g7, e7).max(axis=-1)

    encoded = jnp.transpose(s3, (0, 2, 1))
    return (s1, s2, s3, pts1, pts2, pts3, encoded)


# speedup vs baseline: 2.2688x; 1.7125x over previous
import functools

import jax
import jax.numpy as jnp
from jax.experimental import pallas as pl
from jax.experimental.pallas import tpu as pltpu


def _lrelu(x):
    return jnp.where(x > 0, x, 0.2 * x)


def _square_distance(src, dst):
    d = -2.0 * jnp.matmul(src, jnp.transpose(dst, (0, 2, 1)))
    d = d + jnp.sum(src ** 2, -1)[:, :, None]
    d = d + jnp.sum(dst ** 2, -1)[:, None, :]
    return d


def _knn_idx(k, coor_k, coor_q):
    d = _square_distance(coor_q, coor_k)
    _, idx = jax.lax.top_k(-d, k)
    return idx


def _knn_kernel(q_ref, kt_ref, o_ref, *, nk):
    q = q_ref[0]
    kt = kt_ref[0]
    tq = q.shape[0]
    m = jax.lax.dot_general(q, kt, (((1,), (0,)), ((), ())),
                            preferred_element_type=jnp.float32)
    qx = q[:, 0:1]; qy = q[:, 1:2]; qz = q[:, 2:3]
    kx = kt[0:1, :]; ky = kt[1:2, :]; kz = kt[2:3, :]
    qsq = (qx * qx + qy * qy) + qz * qz
    ksq = (kx * kx + ky * ky) + kz * kz
    d = ((-2.0 * m) + qsq) + ksq
    iota = jax.lax.broadcasted_iota(jnp.int32, (tq, nk), 1)
    cols = []
    for _ in range(16):
        mn = jnp.min(d, axis=1, keepdims=True)
        ij = jnp.min(jnp.where(d == mn, iota, nk), axis=1, keepdims=True)
        cols.append(ij)
        d = jnp.where(iota == ij, jnp.float32(3e38), d)
    o_ref[0] = jnp.concatenate(cols, axis=1)


def _knn16(coor_k, coor_q):
    B, nq, _ = coor_q.shape
    nk = coor_k.shape[1]
    tq = min(nq, 256)
    kT = jnp.transpose(coor_k, (0, 2, 1))
    return pl.pallas_call(
        functools.partial(_knn_kernel, nk=nk),
        grid=(B, nq // tq),
        in_specs=[pl.BlockSpec((1, tq, 3), lambda b, t: (b, t, 0)),
                  pl.BlockSpec((1, 3, nk), lambda b, t: (b, 0, 0))],
        out_specs=pl.BlockSpec((1, tq, 16), lambda b, t: (b, t, 0)),
        out_shape=jax.ShapeDtypeStruct((B, nq, 16), jnp.int32),
    )(coor_q, kT)


def _fps_stage(X, Y, Z, npoint, fi_ref, px_ref, py_ref, pz_ref,
               dists_ref, far_ref):
    B, N = X.shape
    iota = jax.lax.broadcasted_iota(jnp.int32, (B, N), 1)
    oiota = jax.lax.broadcasted_iota(jnp.int32, (B, npoint), 1)

    dists_ref[:, :N] = jnp.full((B, N), 1e10, jnp.float32)
    far_ref[...] = jnp.zeros((B, 128), jnp.int32)

    def body(i, _):
        far = far_ref[:, 0:1]
        slot = oiota == i
        fi_ref[...] = jnp.where(slot, far + jnp.zeros_like(oiota), fi_ref[...])
        sel = iota == far
        cx = jnp.sum(jnp.where(sel, X, 0.0), axis=1, keepdims=True)
        cy = jnp.sum(jnp.where(sel, Y, 0.0), axis=1, keepdims=True)
        cz = jnp.sum(jnp.where(sel, Z, 0.0), axis=1, keepdims=True)
        zf = jnp.zeros((B, npoint), jnp.float32)
        px_ref[...] = jnp.where(slot, cx + zf, px_ref[...])
        py_ref[...] = jnp.where(slot, cy + zf, py_ref[...])
        pz_ref[...] = jnp.where(slot, cz + zf, pz_ref[...])
        dx = X - cx
        dy = Y - cy
        dz = Z - cz
        d = (dx * dx + dz * dz) + dy * dy
        dists = jnp.minimum(dists_ref[:, :N], d)
        dists_ref[:, :N] = dists
        m = jnp.max(dists, axis=1, keepdims=True)
        far_ref[:, 0:1] = jnp.min(jnp.where(dists == m, iota, N),
                                  axis=1, keepdims=True)
        return 0

    jax.lax.fori_loop(0, npoint, body, 0)


def _fps_kernel(x_ref, y_ref, z_ref,
                fi1_ref, p1x_ref, p1y_ref, p1z_ref,
                fi2_ref, p2x_ref, p2y_ref, p2z_ref,
                fi3_ref, p3x_ref, p3y_ref, p3z_ref,
                dists_ref, far_ref):
    _fps_stage(x_ref[...], y_ref[...], z_ref[...], 512,
               fi1_ref, p1x_ref, p1y_ref, p1z_ref, dists_ref, far_ref)
    _fps_stage(p1x_ref[...], p1y_ref[...], p1z_ref[...], 256,
               fi2_ref, p2x_ref, p2y_ref, p2z_ref, dists_ref, far_ref)
    _fps_stage(p2x_ref[...], p2y_ref[...], p2z_ref[...], 128,
               fi3_ref, p3x_ref, p3y_ref, p3z_ref, dists_ref, far_ref)


def _fps_all(x):
    B, N, _ = x.shape
    xT = jnp.transpose(x, (0, 2, 1))
    X, Y, Z = xT[:, 0], xT[:, 1], xT[:, 2]
    outs = pl.pallas_call(
        _fps_kernel,
        out_shape=[
            jax.ShapeDtypeStruct((B, 512), jnp.int32),
            jax.ShapeDtypeStruct((B, 512), jnp.float32),
            jax.ShapeDtypeStruct((B, 512), jnp.float32),
            jax.ShapeDtypeStruct((B, 512), jnp.float32),
            jax.ShapeDtypeStruct((B, 256), jnp.int32),
            jax.ShapeDtypeStruct((B, 256), jnp.float32),
            jax.ShapeDtypeStruct((B, 256), jnp.float32),
            jax.ShapeDtypeStruct((B, 256), jnp.float32),
            jax.ShapeDtypeStruct((B, 128), jnp.int32),
            jax.ShapeDtypeStruct((B, 128), jnp.float32),
            jax.ShapeDtypeStruct((B, 128), jnp.float32),
            jax.ShapeDtypeStruct((B, 128), jnp.float32),
        ],
        scratch_shapes=[
            pltpu.VMEM((B, N), jnp.float32),
            pltpu.VMEM((B, 128), jnp.int32),
        ],
    )(X, Y, Z)
    fi1, p1x, p1y, p1z, fi2, p2x, p2y, p2z, fi3, p3x, p3y, p3z = outs
    pts1 = jnp.stack([p1x, p1y, p1z], axis=-1)
    pts2 = jnp.stack([p2x, p2y, p2z], axis=-1)
    pts3 = jnp.stack([p3x, p3y, p3z], axis=-1)
    return fi1, fi2, fi3, pts1, pts2, pts3


def _graph_feature(x_q, x_k, idx):
    B = x_q.shape[0]
    xk_t = jnp.transpose(x_k, (0, 2, 1))
    g = xk_t[jnp.arange(B)[:, None, None], idx]
    feat = jnp.transpose(g, (0, 3, 1, 2))
    xq = x_q[:, :, :, None]
    return jnp.concatenate([feat - xq, jnp.broadcast_to(xq, feat.shape)], axis=1)


def _conv_gn_lrelu(x, W, gamma, beta, groups=4, eps=1e-5):
    y = jnp.einsum('oi,bink->bonk', W, x)
    B, C, N, K = y.shape
    yg = y.reshape(B, groups, C // groups, N, K)
    m = jnp.mean(yg, axis=(2, 3, 4), keepdims=True)
    v = jnp.var(yg, axis=(2, 3, 4), keepdims=True)
    yg = (yg - m) / jnp.sqrt(v + eps)
    y = yg.reshape(B, C, N, K) * gamma[None, :, None, None] + beta[None, :, None, None]
    return _lrelu(y)


def _stem_kernel(x_ref, w_ref, b_ref, o_ref):
    x = x_ref[0]
    w = w_ref[...]
    b = b_ref[...]
    o_ref[0] = jax.lax.dot_general(
        w, x, (((1,), (0,)), ((), ())),
        preferred_element_type=jnp.float32) + b


def _stem(xyzT, W0, b0):
    B, _, N = xyzT.shape
    return pl.pallas_call(
        _stem_kernel,
        grid=(B,),
        in_specs=[
            pl.BlockSpec((1, 3, N), lambda b: (b, 0, 0)),
            pl.BlockSpec((8, 3), lambda b: (0, 0)),
            pl.BlockSpec((8, 1), lambda b: (0, 0)),
        ],
        out_specs=pl.BlockSpec((1, 8, N), lambda b: (b, 0, 0)),
        out_shape=jax.ShapeDtypeStruct((B, 8, N), jnp.float32),
    )(xyzT, W0, b0[:, None])


def kernel(x, num0, num1, num2, W0, b0, W1, g1, e1, W2, g2, e2, W3, g3, e3,
           W4, g4, e4, W5, g5, e5, W6, g6, e6, W7, g7, e7):
    k = 16
    coor = x
    fi1, fi2, fi3, pts1, pts2, pts3 = _fps_all(coor)
    i0 = _knn16(coor, coor)
    i1 = _knn16(coor, pts1)
    i2 = _knn16(pts1, pts1)
    i3 = _knn16(pts1, pts2)
    i4 = _knn16(pts2, pts2)
    i5 = _knn16(pts2, pts3)
    i6 = _knn16(pts3, pts3)

    xyzT = jnp.transpose(x, (0, 2, 1))
    f = _stem(xyzT, W0, b0)
    f = _conv_gn_lrelu(_graph_feature(f, f, i0), W1, g1, e1).max(axis=-1)
    f_q = jnp.take_along_axis(f, fi1[:, None, :], axis=2)
    f = _conv_gn_lrelu(_graph_feature(f_q, f, i1), W2, g2, e2).max(axis=-1)
    s1 = _conv_gn_lrelu(_graph_feature(f, f, i2), W3, g3, e3).max(axis=-1)
    f_q = jnp.take_along_axis(s1, fi2[:, None, :], axis=2)
    f = _conv_gn_lrelu(_graph_feature(f_q, s1, i3), W4, g4, e4).max(axis=-1)
    s2 = _conv_gn_lrelu(_graph_feature(f, f, i4), W5, g5, e5).max(axis=-1)
    f_q = jnp.take_along_axis(s2, fi3[:, None, :], axis=2)
    f = _conv_gn_lrelu(_graph_feature(f_q, s2, i5), W6, g6, e6).max(axis=-1)
    s3 = _conv_gn_lrelu(_graph_feature(f, f, i6), W7, g7, e7).max(axis=-1)

    encoded = jnp.transpose(s3, (0, 2, 1))
    return (s1, s2, s3, pts1, pts2, pts3, encoded)
